# Initial kernel scaffold; baseline (speedup 1.0000x reference)
import jax
import jax.numpy as jnp
from jax.experimental import pallas as pl


def kernel(z, a):
    raise NotImplementedError("write your pallas kernel here")


# speedup vs baseline: 2.2621x; 2.2621x over previous
import functools

import jax
import jax.numpy as jnp
from jax import lax
from jax.experimental import pallas as pl
from jax.experimental.pallas import tpu as pltpu
from jax.experimental.pallas import tpu_sc as plsc

_ATTR_INDEX = 4


def kernel(z, a):
    B, n_attrs = z.shape
    C = a.shape[1]
    info = plsc.get_sparse_core_info()
    NC, NS, L = info.num_cores, info.num_subcores, info.num_lanes
    NW = NC * NS
    bpw = B // NW
    n_rows = bpw // C
    vecs_per_row = C // L

    a_flat = a.reshape(B * C)
    z_flat = z.reshape(B * n_attrs)

    mesh = plsc.VectorSubcoreMesh(core_axis_name="c", subcore_axis_name="s")

    @functools.partial(
        pl.kernel,
        mesh=mesh,
        out_type=jax.ShapeDtypeStruct((NW * n_rows, C), jnp.float32),
        scratch_types=[
            pltpu.VMEM((n_rows, C), jnp.int32),
            pltpu.VMEM((n_rows, C), jnp.float32),
            pltpu.VMEM((n_rows, C), jnp.int32),
            pltpu.VMEM((n_rows, C), jnp.float32),
            pltpu.SemaphoreType.DMA,
        ],
    )
    def sc_kernel(z_hbm, a_hbm, out_hbm, idxz_v, zcol_v, idxa_v, val_v, sem):
        wid = lax.axis_index("s") * NC + lax.axis_index("c")
        base = wid * bpw

        for j in range(bpw // L):
            rows = lax.iota(jnp.int32, L) + (base + j * L)
            r, o = j // vecs_per_row, (j % vecs_per_row) * L
            idxz_v[r, pl.ds(o, L)] = rows * n_attrs + _ATTR_INDEX

        zcopies = [
            pltpu.async_copy(z_hbm.at[idxz_v.at[r]], zcol_v.at[r], sem)
            for r in range(n_rows)
        ]
        for c in zcopies:
            c.wait()

        for j in range(bpw // L):
            rows = lax.iota(jnp.int32, L) + (base + j * L)
            r, o = j // vecs_per_row, (j % vecs_per_row) * L
            zv = zcol_v[r, pl.ds(o, L)]
            idxa_v[r, pl.ds(o, L)] = rows * C + zv.astype(jnp.int32)

        acopies = [
            pltpu.async_copy(a_hbm.at[idxa_v.at[r]], val_v.at[r], sem)
            for r in range(n_rows)
        ]
        for c in acopies:
            c.wait()

        scale = jnp.full((L,), 0.999, dtype=jnp.float32)
        for j in range(bpw // L):
            r, o = j // vecs_per_row, (j % vecs_per_row) * L
            val_v[r, pl.ds(o, L)] = val_v[r, pl.ds(o, L)] * scale

        pltpu.sync_copy(val_v, out_hbm.at[pl.ds(wid * n_rows, n_rows)])

    return sc_kernel(z_flat, a_flat).reshape(B)


# speedup vs baseline: 2.2841x; 1.0097x over previous
import functools

import jax
import jax.numpy as jnp
from jax import lax
from jax.experimental import pallas as pl
from jax.experimental.pallas import tpu as pltpu
from jax.experimental.pallas import tpu_sc as plsc

_ATTR_INDEX = 4


def kernel(z, a):
    B, n_attrs = z.shape
    C = a.shape[1]
    info = plsc.get_sparse_core_info()
    NC, NS, L = info.num_cores, info.num_subcores, info.num_lanes
    NW = NC * NS
    bpw = B // NW
    n_rows = bpw // C
    vecs_per_row = C // L

    a_flat = a.reshape(B * C)
    z_flat = z.reshape(B * n_attrs)

    mesh = plsc.VectorSubcoreMesh(core_axis_name="c", subcore_axis_name="s")

    @functools.partial(
        pl.kernel,
        mesh=mesh,
        out_type=jax.ShapeDtypeStruct((NW * n_rows, C), jnp.float32),
        scratch_types=[
            pltpu.VMEM((n_rows, C), jnp.int32),
            pltpu.VMEM((n_rows, C), jnp.float32),
            pltpu.VMEM((n_rows, C), jnp.int32),
            pltpu.VMEM((n_rows, C), jnp.float32),
        ]
        + [pltpu.SemaphoreType.DMA] * (2 * n_rows),
    )
    def sc_kernel(z_hbm, a_hbm, out_hbm, idxz_v, zcol_v, idxa_v, val_v, *sems):
        zsem, asem = sems[:n_rows], sems[n_rows:]
        wid = lax.axis_index("s") * NC + lax.axis_index("c")
        base = wid * bpw

        zcopies = []
        for r in range(n_rows):
            for v in range(vecs_per_row):
                rows = lax.iota(jnp.int32, L) + (base + r * C + v * L)
                idxz_v[r, pl.ds(v * L, L)] = rows * n_attrs + _ATTR_INDEX
            zcopies.append(
                pltpu.async_copy(z_hbm.at[idxz_v.at[r]], zcol_v.at[r], zsem[r])
            )

        acopies = []
        for r in range(n_rows):
            zcopies[r].wait()
            for v in range(vecs_per_row):
                rows = lax.iota(jnp.int32, L) + (base + r * C + v * L)
                zv = zcol_v[r, pl.ds(v * L, L)]
                idxa_v[r, pl.ds(v * L, L)] = rows * C + zv.astype(jnp.int32)
            acopies.append(
                pltpu.async_copy(a_hbm.at[idxa_v.at[r]], val_v.at[r], asem[r])
            )

        scale = jnp.full((L,), 0.999, dtype=jnp.float32)
        for r in range(n_rows):
            acopies[r].wait()
            for v in range(vecs_per_row):
                o = v * L
                val_v[r, pl.ds(o, L)] = val_v[r, pl.ds(o, L)] * scale

        pltpu.sync_copy(val_v, out_hbm.at[pl.ds(wid * n_rows, n_rows)])

    return sc_kernel(z_flat, a_flat).reshape(B)


# speedup vs baseline: 2.2847x; 1.0003x over previous
import functools

import jax
import jax.numpy as jnp
from jax import lax
from jax.experimental import pallas as pl
from jax.experimental.pallas import tpu as pltpu
from jax.experimental.pallas import tpu_sc as plsc

_ATTR_INDEX = 4


def kernel(z, a):
    B, n_attrs = z.shape
    C = a.shape[1]
    info = plsc.get_sparse_core_info()
    NC, NS, L = info.num_cores, info.num_subcores, info.num_lanes
    NW = NC * NS
    bpw = B // NW
    n_rows = bpw // C
    vecs_per_row = C // L

    a_flat = a.reshape(B * C)
    z_flat = z.reshape(B * n_attrs)

    mesh = plsc.VectorSubcoreMesh(core_axis_name="c", subcore_axis_name="s")

    @functools.partial(
        pl.kernel,
        mesh=mesh,
        out_type=jax.ShapeDtypeStruct((NW * n_rows, C), jnp.float32),
        scratch_types=[
            pltpu.VMEM((n_rows, C), jnp.int32),
            pltpu.VMEM((n_rows, C), jnp.float32),
            pltpu.VMEM((n_rows, C), jnp.int32),
            pltpu.VMEM((n_rows, C), jnp.float32),
        ]
        + [pltpu.SemaphoreType.DMA] * (3 * n_rows),
    )
    def sc_kernel(z_hbm, a_hbm, out_hbm, idxz_v, zcol_v, idxa_v, val_v, *sems):
        zsem, asem, osem = (
            sems[:n_rows], sems[n_rows:2 * n_rows], sems[2 * n_rows:]
        )
        wid = lax.axis_index("s") * NC + lax.axis_index("c")
        base = wid * bpw

        zcopies = []
        for r in range(n_rows):
            for v in range(vecs_per_row):
                rows = lax.iota(jnp.int32, L) + (base + r * C + v * L)
                idxz_v[r, pl.ds(v * L, L)] = rows * n_attrs + _ATTR_INDEX
            zcopies.append(
                pltpu.async_copy(z_hbm.at[idxz_v.at[r]], zcol_v.at[r], zsem[r])
            )

        acopies = []
        for r in range(n_rows):
            zcopies[r].wait()
            for v in range(vecs_per_row):
                rows = lax.iota(jnp.int32, L) + (base + r * C + v * L)
                zv = zcol_v[r, pl.ds(v * L, L)]
                idxa_v[r, pl.ds(v * L, L)] = rows * C + zv.astype(jnp.int32)
            acopies.append(
                pltpu.async_copy(a_hbm.at[idxa_v.at[r]], val_v.at[r], asem[r])
            )

        scale = jnp.full((L,), 0.999, dtype=jnp.float32)
        ocopies = []
        for r in range(n_rows):
            acopies[r].wait()
            for v in range(vecs_per_row):
                o = v * L
                val_v[r, pl.ds(o, L)] = val_v[r, pl.ds(o, L)] * scale
            ocopies.append(
                pltpu.async_copy(
                    val_v.at[r], out_hbm.at[wid * n_rows + r], osem[r]
                )
            )
        for c in ocopies:
            c.wait()

    return sc_kernel(z_flat, a_flat).reshape(B)
